# Initial kernel scaffold; baseline (speedup 1.0000x reference)
import jax
import jax.numpy as jnp
from jax import lax
from jax.experimental import pallas as pl
from jax.experimental.pallas import tpu as pltpu
from jax.experimental.pallas import tpu_sc as plsc

VOCAB = 100000
HIDDEN = 128
SEQ = 200
BATCH = 4096
FLAT = BATCH * SEQ
EPS = 1e-12

NC, NS = 2, 16
NW = NC * NS
SEQ_PER_W = BATCH // NW
GA = 128
GB = SEQ - GA
NLANE = 16
NV = HIDDEN // NLANE


def _tree_add(vs):
    while len(vs) > 1:
        vs = [a + b for a, b in zip(vs[::2], vs[1::2])] + (
            [vs[-1]] if len(vs) % 2 else [])
    return vs[0]


def _rsqrt16(v):
    i = lax.bitcast_convert_type(v, jnp.int32)
    y = lax.bitcast_convert_type(
        jnp.full((NLANE,), 0x5F3759DF, jnp.int32) - (i >> 1), jnp.float32)
    for _ in range(3):
        y = y * (1.5 - 0.5 * v * y * y)
    return y


def _body(ids_hbm, word_hbm, comb_hbm, gam_hbm, bet_hbm, out_hbm,
          idx_a, idx_b, rows_a, rows_b, out_a, out_b, comb_v, gam_v, bet_v,
          sem):
    wid = lax.axis_index("s") * NC + lax.axis_index("c")
    pltpu.sync_copy(comb_hbm, comb_v)
    pltpu.sync_copy(gam_hbm, gam_v)
    pltpu.sync_copy(bet_hbm, bet_v)

    def ln_block(rows_v, o_v, pos_off, n_rows):
        def quad(q, _):
            r0 = q * 4
            for j in range(4):
                r = r0 + j
                e = []
                for k in range(NV):
                    x = rows_v[r, pl.ds(k * 16, 16)]
                    c = comb_v[pos_off + r, pl.ds(k * 16, 16)]
                    e.append(x + c)
                mean = jnp.sum(_tree_add(e)) * (1.0 / HIDDEN)
                mv = jnp.full((NLANE,), mean, jnp.float32)
                cen = [ek - mv for ek in e]
                var = jnp.sum(_tree_add([ck * ck for ck in cen])) * (1.0 / HIDDEN)
                rv = _rsqrt16(jnp.full((NLANE,), var + EPS, jnp.float32))
                for k in range(NV):
                    g = gam_v[pl.ds(k * 16, 16)]
                    b = bet_v[pl.ds(k * 16, 16)]
                    o_v[r, pl.ds(k * 16, 16)] = cen[k] * rv * g + b
            return 0
        lax.fori_loop(0, n_rows // 4, quad, 0)

    def seq_body(i, _):
        base = (wid * SEQ_PER_W + i) * SEQ
        pltpu.sync_copy(ids_hbm.at[pl.ds(base, GA)], idx_a)
        pltpu.sync_copy(ids_hbm.at[pl.ds(base + GA, GB)], idx_b)
        cp1 = pltpu.async_copy(word_hbm.at[idx_a], rows_a, sem)
        cp2 = pltpu.async_copy(word_hbm.at[idx_b], rows_b, sem)
        cp1.wait()
        cp2.wait()
        ln_block(rows_a, out_a, 0, GA)
        ln_block(rows_b, out_b, GA, GB)
        pltpu.sync_copy(out_a, out_hbm.at[pl.ds(base, GA)])
        pltpu.sync_copy(out_b, out_hbm.at[pl.ds(base + GA, GB)])
        return 0

    lax.fori_loop(0, SEQ_PER_W, seq_body, 0)


def kernel(input_ids, word_table, pos_table, type_table, gamma, beta):
    ids_flat = input_ids.reshape(FLAT).astype(jnp.int32)
    comb = pos_table[:SEQ] + type_table[0][None, :]
    mesh = plsc.VectorSubcoreMesh(core_axis_name="c", subcore_axis_name="s")
    k = pl.kernel(
        _body,
        out_type=jax.ShapeDtypeStruct((FLAT, HIDDEN), jnp.float32),
        mesh=mesh,
        scratch_types=[
            pltpu.VMEM((GA,), jnp.int32),
            pltpu.VMEM((GB,), jnp.int32),
            pltpu.VMEM((GA, HIDDEN), jnp.float32),
            pltpu.VMEM((GB, HIDDEN), jnp.float32),
            pltpu.VMEM((GA, HIDDEN), jnp.float32),
            pltpu.VMEM((GB, HIDDEN), jnp.float32),
            pltpu.VMEM((SEQ, HIDDEN), jnp.float32),
            pltpu.VMEM((HIDDEN,), jnp.float32),
            pltpu.VMEM((HIDDEN,), jnp.float32),
            pltpu.SemaphoreType.DMA,
        ],
    )
    out = k(ids_flat, word_table, comb, gamma, beta)
    return out.reshape(BATCH, SEQ, HIDDEN)


# speedup vs baseline: 1.6649x; 1.6649x over previous
import jax
import jax.numpy as jnp
from jax import lax
from jax.experimental import pallas as pl
from jax.experimental.pallas import tpu as pltpu
from jax.experimental.pallas import tpu_sc as plsc

VOCAB = 100000
HIDDEN = 128
SEQ = 200
BATCH = 4096
FLAT = BATCH * SEQ
EPS = 1e-12

NC, NS = 2, 16
NW = NC * NS
SEQ_PER_W = BATCH // NW
GA = 128
GB = SEQ - GA
NLANE = 16
NV = HIDDEN // NLANE


def _tree_add(vs):
    while len(vs) > 1:
        vs = [a + b for a, b in zip(vs[::2], vs[1::2])] + (
            [vs[-1]] if len(vs) % 2 else [])
    return vs[0]


def _rsqrt16(v):
    i = lax.bitcast_convert_type(v, jnp.int32)
    y = lax.bitcast_convert_type(
        jnp.full((NLANE,), 0x5F3759DF, jnp.int32) - (i >> 1), jnp.float32)
    for _ in range(3):
        y = y * (1.5 - 0.5 * v * y * y)
    return y


def _body(ids_hbm, word_hbm, comb_hbm, gam_hbm, bet_hbm, out_hbm,
          idx_a, idx_b, rows_a, rows_b, out_a, out_b, comb_v, gam_v, bet_v,
          sem):
    wid = lax.axis_index("s") * NC + lax.axis_index("c")
    pltpu.sync_copy(comb_hbm, comb_v)
    pltpu.sync_copy(gam_hbm, gam_v)
    pltpu.sync_copy(bet_hbm, bet_v)

    def ln_block(rows_v, o_v, pos_off, n_rows):
        def quad(q, _):
            r0 = q * 4
            for j in range(4):
                r = r0 + j
                e = []
                for k in range(NV):
                    x = rows_v[r, pl.ds(k * 16, 16)]
                    c = comb_v[pos_off + r, pl.ds(k * 16, 16)]
                    e.append(x + c)
                mean = jnp.sum(_tree_add(e)) * (1.0 / HIDDEN)
                mv = jnp.full((NLANE,), mean, jnp.float32)
                cen = [ek - mv for ek in e]
                var = jnp.sum(_tree_add([ck * ck for ck in cen])) * (1.0 / HIDDEN)
                rv = _rsqrt16(jnp.full((NLANE,), var + EPS, jnp.float32))
                for k in range(NV):
                    g = gam_v[pl.ds(k * 16, 16)]
                    b = bet_v[pl.ds(k * 16, 16)]
                    o_v[r, pl.ds(k * 16, 16)] = cen[k] * rv * g + b
            return 0
        lax.fori_loop(0, n_rows // 4, quad, 0)

    def seq_body(i, _):
        base = (wid * SEQ_PER_W + i) * SEQ
        pltpu.sync_copy(ids_hbm.at[pl.ds(base, GA)], idx_a)
        pltpu.sync_copy(ids_hbm.at[pl.ds(base + GA, GB)], idx_b)
        cp1 = pltpu.async_copy(word_hbm.at[idx_a], rows_a, sem)
        cp2 = pltpu.async_copy(word_hbm.at[idx_b], rows_b, sem)
        cp1.wait()
        cp2.wait()
        ln_block(rows_a, out_a, 0, GA)
        ln_block(rows_b, out_b, GA, GB)
        pltpu.sync_copy(out_a, out_hbm.at[pl.ds(base, GA)])
        pltpu.sync_copy(out_b, out_hbm.at[pl.ds(base + GA, GB)])
        return 0

    lax.fori_loop(0, SEQ_PER_W, seq_body, 0)


def kernel(input_ids, word_table, pos_table, type_table, gamma, beta):
    ids_flat = input_ids.reshape(FLAT).astype(jnp.int32)
    comb = pos_table[:SEQ] + type_table[0][None, :]
    mesh = plsc.VectorSubcoreMesh(core_axis_name="c", subcore_axis_name="s")
    k = pl.kernel(
        _body,
        out_type=jax.ShapeDtypeStruct((FLAT, HIDDEN), jnp.float32),
        mesh=mesh,
        compiler_params=pltpu.CompilerParams(needs_layout_passes=False),
        scratch_types=[
            pltpu.VMEM((GA,), jnp.int32),
            pltpu.VMEM((GB,), jnp.int32),
            pltpu.VMEM((GA, HIDDEN), jnp.float32),
            pltpu.VMEM((GB, HIDDEN), jnp.float32),
            pltpu.VMEM((GA, HIDDEN), jnp.float32),
            pltpu.VMEM((GB, HIDDEN), jnp.float32),
            pltpu.VMEM((SEQ, HIDDEN), jnp.float32),
            pltpu.VMEM((HIDDEN,), jnp.float32),
            pltpu.VMEM((HIDDEN,), jnp.float32),
            pltpu.SemaphoreType.DMA,
        ],
    )
    out = k(ids_flat, word_table, comb, gamma, beta)
    return out.reshape(BATCH, SEQ, HIDDEN)
